# baseline (speedup 1.0000x reference)
import functools

import jax
import jax.numpy as jnp
from jax import lax
from jax.experimental import pallas as pl
from jax.experimental.pallas import tpu as pltpu
from jax.experimental.pallas import tpu_sc as plsc

_D = 64
_NC = 2
_NS = 16
_NW = _NC * _NS
_NPT = 320
_N_PAD = _NW * _NPT
_E_PAD = 344064
_WS = 2048
_FB = 1024
_W = 128
_B_CAP = _E_PAD + _NW * _FB

_MESH = plsc.VectorSubcoreMesh(
    core_axis_name="c", subcore_axis_name="s", num_cores=_NC, num_subcores=_NS)
_SC_PARAMS = pltpu.CompilerParams(needs_layout_passes=False)


def _wid():
    return lax.axis_index("s") * _NC + lax.axis_index("c")


def _lanes():
    return lax.iota(jnp.int32, 16)


def _region_info(cnt_ref, wid):
    start = jnp.int32(0)
    region = jnp.int32(0)
    for k in range(_NW):
        ck = cnt_ref[k, pl.ds(0, 16)][0]
        rk = ((ck >> 10) + 1) << 10
        start = start + jnp.where(wid > k, rk, 0)
        region = region + jnp.where(wid == k, rk, 0)
    return start, region


def _count_body(dst_hbm, cnt_hbm, dwin, out16):
    wid = _wid()
    nbase = wid * _NPT
    acc = jnp.zeros((16,), jnp.int32)

    def win(w, acc):
        pltpu.sync_copy(dst_hbm.at[pl.ds(w * _WS, _WS)], dwin)
        for cc in range(_WS // 16):
            d = dwin[pl.ds(cc * 16, 16)]
            msk = (d >= nbase) & (d < nbase + _NPT)
            acc = acc + jnp.where(msk, 1, 0).astype(jnp.int32)
        return acc

    acc = lax.fori_loop(0, _E_PAD // _WS, win, acc)
    total = jnp.int32(0)
    for k in range(16):
        total = total + acc[k]
    out16[...] = jnp.broadcast_to(total, (16,)).astype(jnp.int32)
    pltpu.sync_copy(out16, cnt_hbm.at[wid])


_count_edges = functools.partial(
    pl.kernel,
    _count_body,
    compiler_params=_SC_PARAMS,
    out_type=jax.ShapeDtypeStruct((_NW, 16), jnp.int32),
    mesh=_MESH,
    scratch_types=[
        pltpu.VMEM((_WS,), jnp.int32),
        pltpu.VMEM((16,), jnp.int32),
    ],
)()


def _bucket_body(dst_hbm, src_hbm, cnt_hbm, bdst_hbm, bsrc_hbm,
                 dwin, swin, cntv, stage_d, stage_s):
    wid = _wid()
    nbase = wid * _NPT
    pltpu.sync_copy(cnt_hbm, cntv)
    start, _ = _region_info(cntv, wid)

    fill = jnp.broadcast_to(nbase, (16,)).astype(jnp.int32)

    def initk(k, c):
        stage_d[pl.ds(k * 16, 16)] = fill
        stage_s[pl.ds(k * 16, 16)] = fill
        return c

    lax.fori_loop(0, _FB // 16 + 2, initk, 0)

    def win(w, carry):
        cl, flushed = carry
        pltpu.sync_copy(dst_hbm.at[pl.ds(w * _WS, _WS)], dwin)
        pltpu.sync_copy(src_hbm.at[pl.ds(w * _WS, _WS)], swin)

        def chunk(cc, carry):
            cl, flushed = carry
            d = dwin[pl.ds(cc * 16, 16)]
            s = swin[pl.ds(cc * 16, 16)]
            msk = (d >= nbase) & (d < nbase + _NPT)
            pop = plsc.all_reduce_population_count(msk)[0]

            def compact_store():
                lane = lax.iota(jnp.int32, 16)
                key = jnp.where(msk, lane, lane + 16)
                safe = jnp.broadcast_to(nbase, (16,)).astype(jnp.int32)
                _, vd = plsc.sort_key_val(key, jnp.where(msk, d, safe))
                _, vs = plsc.sort_key_val(key, jnp.where(msk, s, safe))
                stage_d[pl.ds(cl, 16)] = vd
                stage_s[pl.ds(cl, 16)] = vs

            pl.when(pop > 0)(compact_store)
            cl = cl + pop

            def flush():
                off = pl.multiple_of(start + flushed, 8)
                pltpu.sync_copy(stage_d.at[pl.ds(0, _FB)],
                                bdst_hbm.at[pl.ds(off, _FB)])
                pltpu.sync_copy(stage_s.at[pl.ds(0, _FB)],
                                bsrc_hbm.at[pl.ds(off, _FB)])
                for kk in range(2):
                    stage_d[pl.ds(kk * 16, 16)] = stage_d[pl.ds(_FB + kk * 16, 16)]
                    stage_s[pl.ds(kk * 16, 16)] = stage_s[pl.ds(_FB + kk * 16, 16)]

            do = cl >= _FB
            pl.when(do)(flush)
            cl = jnp.where(do, cl - _FB, cl)
            flushed = jnp.where(do, flushed + _FB, flushed)
            return (cl, flushed)

        return lax.fori_loop(0, _WS // 16, chunk, (cl, flushed))

    cl, flushed = lax.fori_loop(0, _E_PAD // _WS, win,
                                (jnp.int32(0), jnp.int32(0)))
    off = pl.multiple_of(start + flushed, 8)
    pltpu.sync_copy(stage_d.at[pl.ds(0, _FB)],
                    bdst_hbm.at[pl.ds(off, _FB)])
    pltpu.sync_copy(stage_s.at[pl.ds(0, _FB)],
                    bsrc_hbm.at[pl.ds(off, _FB)])


_bucket_edges = functools.partial(
    pl.kernel,
    _bucket_body,
    compiler_params=_SC_PARAMS,
    out_type=(jax.ShapeDtypeStruct((_B_CAP,), jnp.int32),
              jax.ShapeDtypeStruct((_B_CAP,), jnp.int32)),
    mesh=_MESH,
    scratch_types=[
        pltpu.VMEM((_WS,), jnp.int32),
        pltpu.VMEM((_WS,), jnp.int32),
        pltpu.VMEM((_NW, 16), jnp.int32),
        pltpu.VMEM((_FB + 32,), jnp.int32),
        pltpu.VMEM((_FB + 32,), jnp.int32),
    ],
)()


def _edge_gather_body(t_hbm, bdst_hbm, bsrc_hbm, cnt_hbm, p_hbm,
                      cntv, alocal, idxd, idxs0, idxs1, bs0, bs1, po,
                      sem0, sem1):
    wid = _wid()
    nbase = wid * _NPT
    pltpu.sync_copy(cnt_hbm, cntv)
    start, region = _region_info(cntv, wid)
    nw = region >> 7
    pltpu.sync_copy(t_hbm.at[pl.ds(pl.multiple_of(nbase, 8), _NPT)], alocal)

    def issue(k, idxs, bs, sem):
        pltpu.sync_copy(
            bsrc_hbm.at[pl.ds(pl.multiple_of(start + k * _W, 8), _W)], idxs)
        pltpu.async_copy(t_hbm.at[idxs], bs, sem)

    def drain(bs, sem):
        pltpu.make_async_copy(t_hbm.at[pl.ds(0, _W)], bs, sem).wait()

    def compute_store(k, bs):
        pltpu.sync_copy(
            bdst_hbm.at[pl.ds(pl.multiple_of(start + k * _W, 8), _W)], idxd)

        def grp(gi, c):
            idv = idxd[pl.ds(gi * 16, 16)] - nbase
            for r in range(16):
                al = idv[r]
                i = gi * 16 + r
                for g in range(4):
                    a = pl.ds(g * 16, 16)
                    b = pl.ds(_D + g * 16, 16)
                    po[i, a] = jnp.maximum(alocal[al, a] + bs[i, b], 0.0)
            return c

        lax.fori_loop(0, _W // 16, grp, 0)
        pltpu.sync_copy(
            po, p_hbm.at[pl.ds(pl.multiple_of(start + k * _W, 8), _W)])

    issue(0, idxs0, bs0, sem0)

    def body(k, c):
        @pl.when(k + 1 < nw)
        def _():
            kb = k + 1
            pl.when(kb % 2 == 1)(lambda: issue(kb, idxs1, bs1, sem1))
            pl.when(kb % 2 == 0)(lambda: issue(kb, idxs0, bs0, sem0))

        @pl.when(k % 2 == 0)
        def _():
            drain(bs0, sem0)
            compute_store(k, bs0)

        @pl.when(k % 2 == 1)
        def _():
            drain(bs1, sem1)
            compute_store(k, bs1)

        return c

    lax.fori_loop(0, nw, body, 0)


_edge_gather = functools.partial(
    pl.kernel,
    _edge_gather_body,
    compiler_params=_SC_PARAMS,
    out_type=jax.ShapeDtypeStruct((_B_CAP, _D), jnp.float32),
    mesh=_MESH,
    scratch_types=[
        pltpu.VMEM((_NW, 16), jnp.int32),
        pltpu.VMEM((_NPT, 2 * _D), jnp.float32),
        pltpu.VMEM((_W,), jnp.int32),
        pltpu.VMEM((_W,), jnp.int32),
        pltpu.VMEM((_W,), jnp.int32),
        pltpu.VMEM((_W, 2 * _D), jnp.float32),
        pltpu.VMEM((_W, 2 * _D), jnp.float32),
        pltpu.VMEM((_W, _D), jnp.float32),
        pltpu.SemaphoreType.DMA,
        pltpu.SemaphoreType.DMA,
    ],
)()


def _scatter_body(m_hbm, bdst_hbm, cnt_hbm, h_hbm, cntv, idxd, mbuf, acc):
    wid = _wid()
    nbase = wid * _NPT
    pltpu.sync_copy(cnt_hbm, cntv)
    start, region = _region_info(cntv, wid)
    nw = region >> 7

    neg = jnp.full((16,), -3.0e38, jnp.float32)

    def initr(r, c):
        for g in range(4):
            acc[r, pl.ds(g * 16, 16)] = neg
        return c

    lax.fori_loop(0, _NPT, initr, 0, unroll=8)

    def body(k, c):
        off = pl.multiple_of(start + k * _W, 8)
        pltpu.sync_copy(m_hbm.at[pl.ds(off, _W)], mbuf)
        pltpu.sync_copy(bdst_hbm.at[pl.ds(off, _W)], idxd)

        def grp(gi, c2):
            idv = idxd[pl.ds(gi * 16, 16)] - nbase
            for r in range(16):
                al = idv[r]
                i = gi * 16 + r
                for g in range(4):
                    sl = pl.ds(g * 16, 16)
                    acc[al, sl] = jnp.maximum(acc[al, sl], mbuf[i, sl])
            return c2

        lax.fori_loop(0, _W // 16, grp, 0)
        return c

    lax.fori_loop(0, nw, body, 0)
    pltpu.sync_copy(acc, h_hbm.at[pl.ds(pl.multiple_of(nbase, 8), _NPT)])


_scatter_max = functools.partial(
    pl.kernel,
    _scatter_body,
    compiler_params=_SC_PARAMS,
    out_type=jax.ShapeDtypeStruct((_N_PAD, _D), jnp.float32),
    mesh=_MESH,
    scratch_types=[
        pltpu.VMEM((_NW, 16), jnp.int32),
        pltpu.VMEM((_W,), jnp.int32),
        pltpu.VMEM((_W, _D), jnp.float32),
        pltpu.VMEM((_NPT, _D), jnp.float32),
    ],
)()


def _matmul_body(h_ref, w_ref, b_ref, o_ref):
    acc = jnp.dot(h_ref[...], w_ref[...], preferred_element_type=jnp.float32)
    o_ref[...] = (acc + b_ref[...]).astype(o_ref.dtype)


def _matmul(h, w, b):
    return pl.pallas_call(
        _matmul_body,
        out_shape=jax.ShapeDtypeStruct((h.shape[0], w.shape[1]), jnp.float32),
    )(h, w, b[None, :])


def _matmul_rows(h, w, b, blk=2048):
    rows, k = h.shape
    cols = w.shape[1]
    assert rows % blk == 0
    return pl.pallas_call(
        _matmul_body,
        grid=(rows // blk,),
        in_specs=[
            pl.BlockSpec((blk, k), lambda i: (i, 0)),
            pl.BlockSpec((k, cols), lambda i: (0, 0)),
            pl.BlockSpec((1, cols), lambda i: (0, 0)),
        ],
        out_specs=pl.BlockSpec((blk, cols), lambda i: (i, 0)),
        out_shape=jax.ShapeDtypeStruct((rows, cols), jnp.float32),
    )(h, w, b[None, :])


def kernel(x, edge_index, t, Wt, bt, We, be, enc_W1, enc_b1, enc_W2, enc_b2,
           Wfe, bfe, dec_W1, dec_b1, dec_W2, dec_b2, Wfd, bfd):
    n = x.shape[0]
    loops = jnp.arange(n, dtype=jnp.int32)
    e_real = edge_index.shape[1] + n
    pad = _E_PAD - e_real
    src = jnp.concatenate([edge_index[0].astype(jnp.int32), loops,
                           jnp.zeros((pad,), jnp.int32)])
    dst = jnp.concatenate([edge_index[1].astype(jnp.int32), loops,
                           jnp.zeros((pad,), jnp.int32)])

    cnt = _count_edges(dst)
    bdst, bsrc = _bucket_edges(dst, src, cnt)

    freq = jnp.exp(jnp.linspace(-4.0, 4.0, 32))
    emb = jnp.concatenate([jnp.sin(t * freq), jnp.cos(t * freq)], axis=-1)
    t_emb = emb @ Wt + bt
    x_pad = jnp.concatenate(
        [x + t_emb[None, :], jnp.zeros((_N_PAD - n, x.shape[1]), jnp.float32)])
    h = _matmul(x_pad, We, be)

    def layer(h, W1, b1, W2, b2):
        W1a, W1b = W1[:_D], W1[_D:]
        Wcat = jnp.concatenate([W1a - W1b, W1b], axis=1)
        bcat = jnp.concatenate([b1, jnp.zeros((_D,), jnp.float32)])
        T = _matmul(h, Wcat, bcat)
        p = _edge_gather(T, bdst, bsrc, cnt)
        m = _matmul_rows(p, W2, b2)
        return _scatter_max(m, bdst, cnt)

    for i in range(4):
        h = layer(h, enc_W1[i], enc_b1[i], enc_W2[i], enc_b2[i])
    h = _matmul(h, Wfe, bfe)
    for i in range(4):
        h = layer(h, dec_W1[i], dec_b1[i], dec_W2[i], dec_b2[i])
    return _matmul(h, Wfd, bfd)[:n]


# speedup vs baseline: 1.0977x; 1.0977x over previous
import functools

import jax
import jax.numpy as jnp
from jax import lax
from jax.experimental import pallas as pl
from jax.experimental.pallas import tpu as pltpu
from jax.experimental.pallas import tpu_sc as plsc

_D = 64
_NC = 2
_NS = 16
_NW = _NC * _NS
_NPT = 320
_N_PAD = _NW * _NPT
_E_PAD = 344064
_WS = 2048
_FB = 1024
_W = 128
_B_CAP = _E_PAD + _NW * _FB

_MESH = plsc.VectorSubcoreMesh(
    core_axis_name="c", subcore_axis_name="s", num_cores=_NC, num_subcores=_NS)
_SC_PARAMS = pltpu.CompilerParams(needs_layout_passes=False)


def _wid():
    return lax.axis_index("s") * _NC + lax.axis_index("c")


def _lanes():
    return lax.iota(jnp.int32, 16)


def _region_info(cnt_ref, wid):
    start = jnp.int32(0)
    region = jnp.int32(0)
    for k in range(_NW):
        ck = cnt_ref[k, pl.ds(0, 16)][0]
        rk = ((ck >> 10) + 1) << 10
        start = start + jnp.where(wid > k, rk, 0)
        region = region + jnp.where(wid == k, rk, 0)
    return start, region


def _count_body(dst_hbm, cnt_hbm, dwin, out16):
    wid = _wid()
    nbase = wid * _NPT
    acc = jnp.zeros((16,), jnp.int32)

    def win(w, acc):
        pltpu.sync_copy(dst_hbm.at[pl.ds(w * _WS, _WS)], dwin)
        for cc in range(_WS // 16):
            d = dwin[pl.ds(cc * 16, 16)]
            msk = (d >= nbase) & (d < nbase + _NPT)
            acc = acc + jnp.where(msk, 1, 0).astype(jnp.int32)
        return acc

    acc = lax.fori_loop(0, _E_PAD // _WS, win, acc)
    total = jnp.int32(0)
    for k in range(16):
        total = total + acc[k]
    out16[...] = jnp.broadcast_to(total, (16,)).astype(jnp.int32)
    pltpu.sync_copy(out16, cnt_hbm.at[wid])


_count_edges = functools.partial(
    pl.kernel,
    _count_body,
    compiler_params=_SC_PARAMS,
    out_type=jax.ShapeDtypeStruct((_NW, 16), jnp.int32),
    mesh=_MESH,
    scratch_types=[
        pltpu.VMEM((_WS,), jnp.int32),
        pltpu.VMEM((16,), jnp.int32),
    ],
)()


def _bucket_body(dst_hbm, src_hbm, cnt_hbm, bdst_hbm, bsrc_hbm,
                 dwin, swin, cntv, stage_d, stage_s):
    wid = _wid()
    nbase = wid * _NPT
    pltpu.sync_copy(cnt_hbm, cntv)
    start, _ = _region_info(cntv, wid)

    fill = jnp.broadcast_to(nbase, (16,)).astype(jnp.int32)

    def initk(k, c):
        stage_d[pl.ds(k * 16, 16)] = fill
        stage_s[pl.ds(k * 16, 16)] = fill
        return c

    lax.fori_loop(0, _FB // 16 + 2, initk, 0)

    def win(w, carry):
        cl, flushed = carry
        pltpu.sync_copy(dst_hbm.at[pl.ds(w * _WS, _WS)], dwin)
        pltpu.sync_copy(src_hbm.at[pl.ds(w * _WS, _WS)], swin)

        def chunk(cc, carry):
            cl, flushed = carry
            d = dwin[pl.ds(cc * 16, 16)]
            s = swin[pl.ds(cc * 16, 16)]
            msk = (d >= nbase) & (d < nbase + _NPT)
            pop = plsc.all_reduce_population_count(msk)[0]

            def compact_store():
                lane = lax.iota(jnp.int32, 16)
                key = jnp.where(msk, lane, lane + 16)
                safe = jnp.broadcast_to(nbase, (16,)).astype(jnp.int32)
                _, vd = plsc.sort_key_val(key, jnp.where(msk, d, safe))
                _, vs = plsc.sort_key_val(key, jnp.where(msk, s, safe))
                stage_d[pl.ds(cl, 16)] = vd
                stage_s[pl.ds(cl, 16)] = vs

            pl.when(pop > 0)(compact_store)
            cl = cl + pop

            def flush():
                off = pl.multiple_of(start + flushed, 8)
                pltpu.sync_copy(stage_d.at[pl.ds(0, _FB)],
                                bdst_hbm.at[pl.ds(off, _FB)])
                pltpu.sync_copy(stage_s.at[pl.ds(0, _FB)],
                                bsrc_hbm.at[pl.ds(off, _FB)])
                for kk in range(2):
                    stage_d[pl.ds(kk * 16, 16)] = stage_d[pl.ds(_FB + kk * 16, 16)]
                    stage_s[pl.ds(kk * 16, 16)] = stage_s[pl.ds(_FB + kk * 16, 16)]

            do = cl >= _FB
            pl.when(do)(flush)
            cl = jnp.where(do, cl - _FB, cl)
            flushed = jnp.where(do, flushed + _FB, flushed)
            return (cl, flushed)

        return lax.fori_loop(0, _WS // 16, chunk, (cl, flushed))

    cl, flushed = lax.fori_loop(0, _E_PAD // _WS, win,
                                (jnp.int32(0), jnp.int32(0)))
    off = pl.multiple_of(start + flushed, 8)
    pltpu.sync_copy(stage_d.at[pl.ds(0, _FB)],
                    bdst_hbm.at[pl.ds(off, _FB)])
    pltpu.sync_copy(stage_s.at[pl.ds(0, _FB)],
                    bsrc_hbm.at[pl.ds(off, _FB)])


_bucket_edges = functools.partial(
    pl.kernel,
    _bucket_body,
    compiler_params=_SC_PARAMS,
    out_type=(jax.ShapeDtypeStruct((_B_CAP,), jnp.int32),
              jax.ShapeDtypeStruct((_B_CAP,), jnp.int32)),
    mesh=_MESH,
    scratch_types=[
        pltpu.VMEM((_WS,), jnp.int32),
        pltpu.VMEM((_WS,), jnp.int32),
        pltpu.VMEM((_NW, 16), jnp.int32),
        pltpu.VMEM((_FB + 32,), jnp.int32),
        pltpu.VMEM((_FB + 32,), jnp.int32),
    ],
)()


def _edge_gather_body(t_hbm, bdst_hbm, bsrc_hbm, cnt_hbm, p_hbm,
                      cntv, alocal, idxd, idxs, bs0, bs1, po,
                      sem0, sem1):
    wid = _wid()
    nbase = wid * _NPT
    pltpu.sync_copy(cnt_hbm, cntv)
    start, region = _region_info(cntv, wid)
    nsw = region >> 10
    pltpu.sync_copy(t_hbm.at[pl.ds(pl.multiple_of(nbase, 8), _NPT)], alocal)

    def issue(sw, w, bs, sem):
        pltpu.async_copy(t_hbm.at[idxs.at[pl.ds((w % 8) * _W, _W)]], bs, sem)

    def drain(bs, sem):
        pltpu.make_async_copy(t_hbm.at[pl.ds(0, _W)], bs, sem).wait()

    def load_idx(sw):
        off = pl.multiple_of(start + sw * 1024, 8)
        pltpu.sync_copy(bdst_hbm.at[pl.ds(off, 1024)], idxd)
        pltpu.sync_copy(bsrc_hbm.at[pl.ds(off, 1024)], idxs)

    def compute_store(sw, w, bs):
        def grp(gi, c):
            idv = idxd[pl.ds((w % 8) * _W + gi * 16, 16)] - nbase
            for r in range(16):
                al = idv[r]
                i = gi * 16 + r
                for g in range(4):
                    a = pl.ds(g * 16, 16)
                    b = pl.ds(_D + g * 16, 16)
                    po[i, a] = jnp.maximum(alocal[al, a] + bs[i, b], 0.0)
            return c

        lax.fori_loop(0, _W // 16, grp, 0)
        pltpu.sync_copy(
            po, p_hbm.at[pl.ds(pl.multiple_of(start + sw * 1024 + (w % 8) * _W, 8),
                               _W)])

    def super_body(sw, c):
        load_idx(sw)
        issue(sw, 0, bs0, sem0)
        for w in range(8):
            if w < 7:
                issue(sw, w + 1, (bs1, bs0)[w % 2], (sem1, sem0)[w % 2])
            drain((bs0, bs1)[w % 2], (sem0, sem1)[w % 2])
            compute_store(sw, w, (bs0, bs1)[w % 2])
        return c

    lax.fori_loop(0, nsw, super_body, 0)


_edge_gather = functools.partial(
    pl.kernel,
    _edge_gather_body,
    compiler_params=_SC_PARAMS,
    out_type=jax.ShapeDtypeStruct((_B_CAP, _D), jnp.float32),
    mesh=_MESH,
    scratch_types=[
        pltpu.VMEM((_NW, 16), jnp.int32),
        pltpu.VMEM((_NPT, 2 * _D), jnp.float32),
        pltpu.VMEM((1024,), jnp.int32),
        pltpu.VMEM((1024,), jnp.int32),
        pltpu.VMEM((_W, 2 * _D), jnp.float32),
        pltpu.VMEM((_W, 2 * _D), jnp.float32),
        pltpu.VMEM((_W, _D), jnp.float32),
        pltpu.SemaphoreType.DMA,
        pltpu.SemaphoreType.DMA,
    ],
)()


def _scatter_body(m_hbm, bdst_hbm, cnt_hbm, h_hbm,
                  cntv, idxd, mb0, mb1, acc, sem0, sem1):
    wid = _wid()
    nbase = wid * _NPT
    pltpu.sync_copy(cnt_hbm, cntv)
    start, region = _region_info(cntv, wid)
    nsw = region >> 10

    neg = jnp.full((16,), -3.0e38, jnp.float32)

    def initr(r, c):
        for g in range(4):
            acc[r, pl.ds(g * 16, 16)] = neg
        return c

    lax.fori_loop(0, _NPT, initr, 0, unroll=8)

    def issue(sw, w, mb, sem):
        off = pl.multiple_of(start + sw * 1024 + (w % 8) * _W, 8)
        pltpu.async_copy(m_hbm.at[pl.ds(off, _W)], mb, sem)

    def drain(mb, sem):
        pltpu.make_async_copy(m_hbm.at[pl.ds(0, _W)], mb, sem).wait()

    def rmw(w, mb):
        def grp(gi, c2):
            idv = idxd[pl.ds((w % 8) * _W + gi * 16, 16)] - nbase
            for r in range(16):
                al = idv[r]
                i = gi * 16 + r
                for g in range(4):
                    sl = pl.ds(g * 16, 16)
                    acc[al, sl] = jnp.maximum(acc[al, sl], mb[i, sl])
            return c2

        lax.fori_loop(0, _W // 16, grp, 0)

    def super_body(sw, c):
        off = pl.multiple_of(start + sw * 1024, 8)
        pltpu.sync_copy(bdst_hbm.at[pl.ds(off, 1024)], idxd)
        issue(sw, 0, mb0, sem0)
        for w in range(8):
            if w < 7:
                issue(sw, w + 1, (mb1, mb0)[w % 2], (sem1, sem0)[w % 2])
            drain((mb0, mb1)[w % 2], (sem0, sem1)[w % 2])
            rmw(w, (mb0, mb1)[w % 2])
        return c

    lax.fori_loop(0, nsw, super_body, 0)
    pltpu.sync_copy(acc, h_hbm.at[pl.ds(pl.multiple_of(nbase, 8), _NPT)])


_scatter_max = functools.partial(
    pl.kernel,
    _scatter_body,
    compiler_params=_SC_PARAMS,
    out_type=jax.ShapeDtypeStruct((_N_PAD, _D), jnp.float32),
    mesh=_MESH,
    scratch_types=[
        pltpu.VMEM((_NW, 16), jnp.int32),
        pltpu.VMEM((1024,), jnp.int32),
        pltpu.VMEM((_W, _D), jnp.float32),
        pltpu.VMEM((_W, _D), jnp.float32),
        pltpu.VMEM((_NPT, _D), jnp.float32),
        pltpu.SemaphoreType.DMA,
        pltpu.SemaphoreType.DMA,
    ],
)()


def _matmul_body(h_ref, w_ref, b_ref, o_ref):
    acc = jnp.dot(h_ref[...], w_ref[...], preferred_element_type=jnp.float32)
    o_ref[...] = (acc + b_ref[...]).astype(o_ref.dtype)


def _matmul(h, w, b):
    return pl.pallas_call(
        _matmul_body,
        out_shape=jax.ShapeDtypeStruct((h.shape[0], w.shape[1]), jnp.float32),
    )(h, w, b[None, :])


def _matmul_rows(h, w, b, blk=2048):
    rows, k = h.shape
    cols = w.shape[1]
    assert rows % blk == 0
    return pl.pallas_call(
        _matmul_body,
        grid=(rows // blk,),
        in_specs=[
            pl.BlockSpec((blk, k), lambda i: (i, 0)),
            pl.BlockSpec((k, cols), lambda i: (0, 0)),
            pl.BlockSpec((1, cols), lambda i: (0, 0)),
        ],
        out_specs=pl.BlockSpec((blk, cols), lambda i: (i, 0)),
        out_shape=jax.ShapeDtypeStruct((rows, cols), jnp.float32),
    )(h, w, b[None, :])


def kernel(x, edge_index, t, Wt, bt, We, be, enc_W1, enc_b1, enc_W2, enc_b2,
           Wfe, bfe, dec_W1, dec_b1, dec_W2, dec_b2, Wfd, bfd):
    n = x.shape[0]
    loops = jnp.arange(n, dtype=jnp.int32)
    e_real = edge_index.shape[1] + n
    pad = _E_PAD - e_real
    src = jnp.concatenate([edge_index[0].astype(jnp.int32), loops,
                           jnp.zeros((pad,), jnp.int32)])
    dst = jnp.concatenate([edge_index[1].astype(jnp.int32), loops,
                           jnp.zeros((pad,), jnp.int32)])

    cnt = _count_edges(dst)
    bdst, bsrc = _bucket_edges(dst, src, cnt)

    freq = jnp.exp(jnp.linspace(-4.0, 4.0, 32))
    emb = jnp.concatenate([jnp.sin(t * freq), jnp.cos(t * freq)], axis=-1)
    t_emb = emb @ Wt + bt
    x_pad = jnp.concatenate(
        [x + t_emb[None, :], jnp.zeros((_N_PAD - n, x.shape[1]), jnp.float32)])
    h = _matmul(x_pad, We, be)

    def layer(h, W1, b1, W2, b2):
        W1a, W1b = W1[:_D], W1[_D:]
        Wcat = jnp.concatenate([W1a - W1b, W1b], axis=1)
        bcat = jnp.concatenate([b1, jnp.zeros((_D,), jnp.float32)])
        T = _matmul(h, Wcat, bcat)
        p = _edge_gather(T, bdst, bsrc, cnt)
        m = _matmul_rows(p, W2, b2)
        return _scatter_max(m, bdst, cnt)

    for i in range(4):
        h = layer(h, enc_W1[i], enc_b1[i], enc_W2[i], enc_b2[i])
    h = _matmul(h, Wfe, bfe)
    for i in range(4):
        h = layer(h, dec_W1[i], dec_b1[i], dec_W2[i], dec_b2[i])
    return _matmul(h, Wfd, bfd)[:n]
